# baseline (speedup 1.0000x reference)
import jax
import jax.numpy as jnp
from jax import lax
from jax.experimental import pallas as pl
from jax.experimental.pallas import tpu as pltpu
from jax.experimental.pallas import tpu_sc as plsc

NUM_ROWS_TABLE = 9
NUM_ROWS_OUT = 8
D = 8197120
NW = 32
QUARTERS = NW // NUM_ROWS_OUT
QUARTER = D // QUARTERS
NBUF = 4
CW = 30720
NFULL = QUARTER // CW
TAIL = QUARTER - NFULL * CW
assert QUARTER % 128 == 0 and CW % 128 == 0 and TAIL % 128 == 0


def _body(ids_hbm, table_hbm, out_hbm, ids_v, idx_v, *bufsem):
    w = lax.axis_index("c") * 16 + lax.axis_index("s")
    r = w // QUARTERS
    q = w % QUARTERS

    pltpu.sync_copy(ids_hbm, ids_v)
    rvec = jnp.full((16,), 0, jnp.int32) + r
    idx_v[...] = plsc.load_gather(ids_v, [rvec])

    col0 = pl.multiple_of(q * QUARTER, 128)

    bufs = bufsem[:NBUF]
    sin = bufsem[NBUF:2 * NBUF]
    sout = bufsem[2 * NBUF:]
    sizes = [CW] * NFULL + ([TAIL] if TAIL else [])
    offs = [k * CW for k in range(len(sizes))]

    def start_in(k):
        return pltpu.async_copy(
            table_hbm.at[idx_v.at[pl.ds(0, 1)],
                         pl.ds(col0 + offs[k], sizes[k])],
            bufs[k % NBUF].at[:, pl.ds(0, sizes[k])], sin[k % NBUF])

    def start_out(k):
        return pltpu.async_copy(
            bufs[k % NBUF].at[:, pl.ds(0, sizes[k])],
            out_hbm.at[pl.ds(r, 1), pl.ds(col0 + offs[k], sizes[k])],
            sout[k % NBUF])

    n = len(sizes)
    h_in = {k: start_in(k) for k in range(min(NBUF, n))}
    h_out = {}
    for k in range(n):
        h_in[k].wait()
        h_out[k] = start_out(k)
        if k + NBUF < n:
            h_out[k].wait()
            h_in[k + NBUF] = start_in(k + NBUF)
    for k in range(max(0, n - NBUF), n):
        h_out[k].wait()


@jax.jit
def kernel(tile_ids, table):
    mesh = plsc.VectorSubcoreMesh(core_axis_name="c", subcore_axis_name="s")
    run = pl.kernel(
        _body,
        out_type=jax.ShapeDtypeStruct((NUM_ROWS_OUT, D), jnp.float32),
        mesh=mesh,
        compiler_params=pltpu.CompilerParams(needs_layout_passes=False),
        scratch_types=[
            pltpu.VMEM((NUM_ROWS_OUT,), jnp.int32),
            pltpu.VMEM((16,), jnp.int32),
            *[pltpu.VMEM((1, CW), jnp.float32) for _ in range(NBUF)],
            *[pltpu.SemaphoreType.DMA for _ in range(2 * NBUF)],
        ],
    )
    return run(tile_ids.astype(jnp.int32), table)
